# baseline (device time: 17718 ns/iter reference)
import os

import jax
import jax.numpy as jnp
from jax import lax
from jax.experimental import pallas as pl
from jax.experimental.pallas import tpu as pltpu

_PEER_OFFSETS = ((0, 1), (1, 0), (1, 1))

_KVAR = os.environ.get("KVAR", "")


def kernel(x, W, labels):
    T, D = x.shape
    _, V = W.shape
    half = V // 2
    CH = 1024
    NCH = half // CH
    RB = 4
    RR = D // RB

    labels2d = labels.reshape(T, 1)

    def body(x_ref, w_hbm, lab_ref, out_ref,
             wbuf, mybuf, peerstats, wsems, send_sems, recv_sems):
        my_x = lax.axis_index("x")
        my_y = lax.axis_index("y")

        base = my_x * half

        def w_dma(c, r):
            return pltpu.make_async_copy(
                w_hbm.at[pl.ds(r * RR, RR), pl.ds(base + c * CH, CH)],
                wbuf.at[c, pl.ds(r * RR, RR), :],
                wsems.at[c * RB + r],
            )

        for c in range(NCH):
            for r in range(RB):
                w_dma(c, r).start()

        barrier_sem = pltpu.get_barrier_semaphore()
        for dx, dy in _PEER_OFFSETS:
            px = (1 - my_x) if dx else my_x
            py = (1 - my_y) if dy else my_y
            pl.semaphore_signal(
                barrier_sem, inc=1,
                device_id=(px, py), device_id_type=pl.DeviceIdType.MESH,
            )
        pl.semaphore_wait(barrier_sem, 3)

        xb = x_ref[:].astype(jnp.bfloat16)
        lab = lab_ref[:]
        m_run = jnp.full((T, 1), -1e30, jnp.float32)
        s_run = jnp.zeros((T, 1), jnp.float32)
        t_run = jnp.zeros((T, 1), jnp.float32)

        for c in range(NCH):
            for r in range(RB):
                w_dma(c, r).wait()
            if _KVAR == "nomatmul":
                logits = wbuf[c, 0:T, :]
            else:
                wb = wbuf[c].astype(jnp.bfloat16)
                logits = jnp.dot(xb, wb, preferred_element_type=jnp.float32)
            if _KVAR in ("nosoftmax", "nomatmul"):
                s_run = s_run + jnp.sum(logits, axis=1, keepdims=True)
                continue
            m_c = jnp.max(logits, axis=1, keepdims=True)
            m_new = jnp.maximum(m_run, m_c)
            s_run = s_run * jnp.exp(m_run - m_new) + jnp.sum(
                jnp.exp(logits - m_new), axis=1, keepdims=True)
            glob_off = my_y * V + base + c * CH
            col = lax.broadcasted_iota(jnp.int32, (T, CH), 1) + glob_off
            t_run = t_run + jnp.sum(
                jnp.where(col == lab, logits, 0.0), axis=1, keepdims=True)
            m_run = m_new

        mybuf[0:1, :] = m_run.T
        mybuf[1:2, :] = s_run.T
        mybuf[2:3, :] = t_run.T

        if _KVAR == "nocomm":
            out_ref[:, :] = mybuf[0:1, :] + mybuf[1:2, :] + mybuf[2:3, :]
            return

        rdmas = []
        for k, (dx, dy) in enumerate(_PEER_OFFSETS):
            px = (1 - my_x) if dx else my_x
            py = (1 - my_y) if dy else my_y
            r = pltpu.make_async_remote_copy(
                src_ref=mybuf,
                dst_ref=peerstats.at[k],
                send_sem=send_sems.at[k],
                recv_sem=recv_sems.at[k],
                device_id=(px, py),
                device_id_type=pl.DeviceIdType.MESH,
            )
            r.start()
            rdmas.append(r)
        for r in rdmas:
            r.wait()

        ms = [mybuf[0:1, :]] + [peerstats[k, 0:1, :] for k in range(3)]
        ss = [mybuf[1:2, :]] + [peerstats[k, 1:2, :] for k in range(3)]
        ts = [mybuf[2:3, :]] + [peerstats[k, 2:3, :] for k in range(3)]
        m = jnp.maximum(jnp.maximum(ms[0], ms[1]), jnp.maximum(ms[2], ms[3]))
        s = sum(s_i * jnp.exp(m_i - m) for s_i, m_i in zip(ss, ms))
        t = ts[0] + ts[1] + ts[2] + ts[3]
        out_ref[:, :] = m + jnp.log(s) - t

    out = pl.pallas_call(
        body,
        out_shape=jax.ShapeDtypeStruct((1, T), jnp.float32),
        in_specs=[
            pl.BlockSpec(memory_space=pltpu.VMEM),
            pl.BlockSpec(memory_space=pl.ANY),
            pl.BlockSpec(memory_space=pltpu.VMEM),
        ],
        out_specs=pl.BlockSpec(memory_space=pltpu.VMEM),
        scratch_shapes=[
            pltpu.VMEM((NCH, D, CH), jnp.float32),
            pltpu.VMEM((3, T), jnp.float32),
            pltpu.VMEM((3, 3, T), jnp.float32),
            pltpu.SemaphoreType.DMA((NCH * RB,)),
            pltpu.SemaphoreType.DMA((3,)),
            pltpu.SemaphoreType.DMA((3,)),
        ],
        compiler_params=pltpu.CompilerParams(collective_id=0),
    )(x, W, labels2d)
    return out.reshape(T)


# device time: 9092 ns/iter; 1.9487x vs baseline; 1.9487x over previous
import os

import jax
import jax.numpy as jnp
from jax import lax
from jax.experimental import pallas as pl
from jax.experimental.pallas import tpu as pltpu

_PEER_OFFSETS = ((0, 1), (1, 0), (1, 1))

_KVAR = os.environ.get("KVAR", "")


def kernel(x, W, labels):
    T, D = x.shape
    _, V = W.shape
    half = V // 2
    CH = 1024
    NCH = half // CH
    RB = 4
    RR = D // RB

    labels2d = labels.reshape(T, 1)

    def body(x_ref, w_hbm, lab_ref, out_ref,
             wbuf, mybuf, peerstats, wsems, send_sems, recv_sems):
        my_x = lax.axis_index("x")
        my_y = lax.axis_index("y")

        base = my_x * half

        def w_dma(c, r):
            return pltpu.make_async_copy(
                w_hbm.at[pl.ds(r * RR, RR), pl.ds(base + c * CH, CH)],
                wbuf.at[c, pl.ds(r * RR, RR), :],
                wsems.at[c * RB + r],
            )

        for c in range(NCH):
            for r in range(RB):
                w_dma(c, r).start()

        if _KVAR == "dmaonly":
            for c in range(NCH):
                for r in range(RB):
                    w_dma(c, r).wait()
            out_ref[:, :] = wbuf[0, 0:1, 0:T]
            return

        barrier_sem = pltpu.get_barrier_semaphore()
        for dx, dy in _PEER_OFFSETS:
            px = (1 - my_x) if dx else my_x
            py = (1 - my_y) if dy else my_y
            pl.semaphore_signal(
                barrier_sem, inc=1,
                device_id=(px, py), device_id_type=pl.DeviceIdType.MESH,
            )
        pl.semaphore_wait(barrier_sem, 3)

        xb = x_ref[:].astype(jnp.bfloat16)
        lab = lab_ref[:]
        m_run = jnp.full((T, 1), -1e30, jnp.float32)
        s_run = jnp.zeros((T, 1), jnp.float32)
        t_run = jnp.zeros((T, 1), jnp.float32)

        for c in range(NCH):
            for r in range(RB):
                w_dma(c, r).wait()
            if _KVAR == "nomatmul":
                logits = wbuf[c, 0:T, :]
            else:
                wb = wbuf[c].astype(jnp.bfloat16)
                logits = jnp.dot(xb, wb, preferred_element_type=jnp.float32)
            if _KVAR in ("nosoftmax", "nomatmul"):
                s_run = s_run + jnp.sum(logits, axis=1, keepdims=True)
                continue
            m_c = jnp.max(logits, axis=1, keepdims=True)
            m_new = jnp.maximum(m_run, m_c)
            s_run = s_run * jnp.exp(m_run - m_new) + jnp.sum(
                jnp.exp(logits - m_new), axis=1, keepdims=True)
            glob_off = my_y * V + base + c * CH
            col = lax.broadcasted_iota(jnp.int32, (T, CH), 1) + glob_off
            t_run = t_run + jnp.sum(
                jnp.where(col == lab, logits, 0.0), axis=1, keepdims=True)
            m_run = m_new

        mybuf[0:1, :] = m_run.T
        mybuf[1:2, :] = s_run.T
        mybuf[2:3, :] = t_run.T

        if _KVAR == "nocomm":
            out_ref[:, :] = mybuf[0:1, :] + mybuf[1:2, :] + mybuf[2:3, :]
            return

        rdmas = []
        for k, (dx, dy) in enumerate(_PEER_OFFSETS):
            px = (1 - my_x) if dx else my_x
            py = (1 - my_y) if dy else my_y
            r = pltpu.make_async_remote_copy(
                src_ref=mybuf,
                dst_ref=peerstats.at[k],
                send_sem=send_sems.at[k],
                recv_sem=recv_sems.at[k],
                device_id=(px, py),
                device_id_type=pl.DeviceIdType.MESH,
            )
            r.start()
            rdmas.append(r)
        for r in rdmas:
            r.wait()

        ms = [mybuf[0:1, :]] + [peerstats[k, 0:1, :] for k in range(3)]
        ss = [mybuf[1:2, :]] + [peerstats[k, 1:2, :] for k in range(3)]
        ts = [mybuf[2:3, :]] + [peerstats[k, 2:3, :] for k in range(3)]
        m = jnp.maximum(jnp.maximum(ms[0], ms[1]), jnp.maximum(ms[2], ms[3]))
        s = sum(s_i * jnp.exp(m_i - m) for s_i, m_i in zip(ss, ms))
        t = ts[0] + ts[1] + ts[2] + ts[3]
        out_ref[:, :] = m + jnp.log(s) - t

    out = pl.pallas_call(
        body,
        out_shape=jax.ShapeDtypeStruct((1, T), jnp.float32),
        in_specs=[
            pl.BlockSpec(memory_space=pltpu.VMEM),
            pl.BlockSpec(memory_space=pl.ANY),
            pl.BlockSpec(memory_space=pltpu.VMEM),
        ],
        out_specs=pl.BlockSpec(memory_space=pltpu.VMEM),
        scratch_shapes=[
            pltpu.VMEM((NCH, D, CH), jnp.float32),
            pltpu.VMEM((3, T), jnp.float32),
            pltpu.VMEM((3, 3, T), jnp.float32),
            pltpu.SemaphoreType.DMA((NCH * RB,)),
            pltpu.SemaphoreType.DMA((3,)),
            pltpu.SemaphoreType.DMA((3,)),
        ],
        compiler_params=pltpu.CompilerParams(
            collective_id=None if _KVAR == "dmaonly" else 0),
    )(x, W, labels2d)
    return out.reshape(T)
